# baseline (speedup 1.0000x reference)
import jax
import jax.numpy as jnp
from jax import lax
from jax.experimental import pallas as pl
from jax.experimental.pallas import tpu as pltpu

N_BUF = 19999
N_PAD = 20000
DIM = 128
CB = 256
G8 = 8


def _al8(i):
    return pl.multiple_of((i >> 3) << 3, 8)


def _tree_lstm_kernel(ps, ls, rs, roots, leaf, wt, bb,
                      hout, rootout,
                      hc, xl, xr, yb, zv, ll, lr, lvl, off, sp, sl, sr, sem):
    E = ps.shape[0]
    NRES = leaf.shape[0]
    f32 = jnp.float32
    sub_iota = lax.broadcasted_iota(jnp.int32, (G8, 1), 0)

    def pick_row(tile, idx):
        sh = (G8 - (idx & 7)) & 7
        return pltpu.roll(tile, sh, 0)[0:1, :]

    zv[...] = jnp.zeros((N_BUF,), jnp.int32)
    cp_ll = pltpu.make_async_copy(zv, ll, sem)
    cp_lr = pltpu.make_async_copy(zv, lr, sem)
    off_n = (E + 2 + 127) // 128 * 128
    cp_off = pltpu.make_async_copy(zv.at[pl.ds(0, off_n)], off, sem)
    cp_ll.start()
    cp_lr.start()
    cp_off.start()

    hc[...] = jnp.zeros((N_PAD, 2 * DIM), f32)
    hc[0:NRES, 0:DIM] = leaf[...]

    cp_ll.wait()
    cp_lr.wait()
    cp_off.wait()

    def edge_level(e):
        le = ls[e]
        re_ = rs[e]
        pe = ps[e]
        lv = jnp.maximum(jnp.maximum(ll[le], ll[re_]) + 1,
                         jnp.maximum(ll[pe], lr[pe]))
        lvl[e] = lv
        lr[le] = jnp.maximum(lr[le], lv)
        lr[re_] = jnp.maximum(lr[re_], lv)
        ll[pe] = lv
        off[lv] = off[lv] + 1
        return lv

    UNR = 12

    def pass1(i, nlev):
        e = i * UNR
        m = nlev
        for u in range(UNR):
            m = jnp.maximum(m, edge_level(e + u))
        return m
    nlev = lax.fori_loop(0, E // UNR, pass1, jnp.int32(0))
    for e_tail in range((E // UNR) * UNR, E):
        nlev = jnp.maximum(nlev, edge_level(e_tail))

    def pass2(L, acc):
        cnt = off[L]
        off[L] = acc
        return acc + cnt
    lax.fori_loop(1, nlev + 1, pass2, jnp.int32(0))

    def edge_place(e):
        lv = lvl[e]
        pos = off[lv]
        off[lv] = pos + 1
        sp[pos] = ps[e]
        sl[pos] = ls[e]
        sr[pos] = rs[e]

    def pass3(i, c):
        e = i * UNR
        for u in range(UNR):
            edge_place(e + u)
        return c
    lax.fori_loop(0, E // UNR, pass3, 0)
    for e_tail in range((E // UNR) * UNR, E):
        edge_place(e_tail)

    def level_body(L, c):
        start = off[L - 1]
        end = off[L]
        nchunks = (end - start + CB - 1) // CB

        def chunk_body(ck, c2):
            base = start + ck * CB
            ngroups = (jnp.minimum(CB, end - base) + G8 - 1) // G8

            def gather_group(g, c3):
                gb = pl.multiple_of(g * G8, G8)
                rows_l = []
                rows_r = []
                for m in range(G8):
                    pos = base + gb + m
                    ok = pos < end
                    il = jnp.where(ok, sl[pos], 0)
                    ir = jnp.where(ok, sr[pos], 0)
                    rows_l.append(pick_row(hc[pl.ds(_al8(il), G8), :], il))
                    rows_r.append(pick_row(hc[pl.ds(_al8(ir), G8), :], ir))
                xl[pl.ds(gb, G8), :] = jnp.concatenate(rows_l, axis=0)
                xr[pl.ds(gb, G8), :] = jnp.concatenate(rows_r, axis=0)
                return c3
            lax.fori_loop(0, ngroups, gather_group, 0)

            xlv = xl[...]
            xrv = xr[...]
            comb = jnp.concatenate([xlv[:, 0:DIM], xrv[:, 0:DIM]], axis=1)
            out = jnp.dot(comb, wt[...], preferred_element_type=f32) + bb[...]
            i_g = jax.nn.sigmoid(out[:, 0:DIM])
            f_l = jax.nn.sigmoid(out[:, DIM:2 * DIM])
            f_r = jax.nn.sigmoid(out[:, 2 * DIM:3 * DIM])
            o_g = jax.nn.sigmoid(out[:, 3 * DIM:4 * DIM])
            g_g = jnp.tanh(out[:, 4 * DIM:5 * DIM])
            c_new = f_l * xlv[:, DIM:2 * DIM] + f_r * xrv[:, DIM:2 * DIM] + i_g * g_g
            h_new = o_g * jnp.tanh(c_new)
            yb[...] = jnp.concatenate([h_new, c_new], axis=1)

            def scatter_group(g, c3):
                gb = pl.multiple_of(g * G8, G8)
                yt = yb[pl.ds(gb, G8), :]
                for m in range(G8):
                    pos = base + gb + m
                    ok = pos < end
                    ip = jnp.where(ok, sp[pos], 0)
                    tb = _al8(ip)
                    tile = hc[pl.ds(tb, G8), :]
                    sel = jnp.logical_and(sub_iota == (ip & 7), ok)
                    row = jnp.broadcast_to(yt[m:m + 1, :], (G8, 2 * DIM))
                    hc[pl.ds(tb, G8), :] = jnp.where(sel, row, tile)
                return c3
            lax.fori_loop(0, ngroups, scatter_group, 0)
            return c2
        lax.fori_loop(0, nchunks, chunk_body, 0)
        return c
    lax.fori_loop(1, nlev + 1, level_body, 0)

    acc = jnp.zeros((1, DIM), f32)
    for k in range(4):
        rt = roots[k]
        acc = acc + pick_row(hc[pl.ds(_al8(rt), G8), 0:DIM], rt)
    rootout[:, 0:DIM] = jnp.broadcast_to(acc * 0.25, (G8, DIM))
    rootout[:, DIM:2 * DIM] = jnp.zeros((G8, DIM), f32)
    hout[:, 0:DIM] = hc[0:NRES, 0:DIM]
    hout[:, DIM:2 * DIM] = jnp.zeros((NRES, DIM), f32)


def kernel(leaf_emb, forest_edges, n_res, forest_roots, W_up, b_up, W_down, b_down):
    f32 = jnp.float32
    NRES, dim = leaf_emb.shape
    E = forest_edges.shape[0]

    max_node = jnp.maximum(jnp.max(forest_edges[:, 0]), n_res - 1)
    p_arr = forest_edges[:, 0]
    l_arr = jnp.minimum(forest_edges[:, 1], max_node)
    r_arr = jnp.minimum(forest_edges[:, 2], max_node)
    roots_c = jnp.minimum(forest_roots, max_node).astype(jnp.int32)

    wt = jnp.transpose(W_up)
    bb = b_up.reshape(1, 5 * dim)

    hout, rootout = pl.pallas_call(
        _tree_lstm_kernel,
        out_shape=[
            jax.ShapeDtypeStruct((NRES, 2 * dim), f32),
            jax.ShapeDtypeStruct((G8, 2 * dim), f32),
        ],
        in_specs=[
            pl.BlockSpec(memory_space=pltpu.SMEM),
            pl.BlockSpec(memory_space=pltpu.SMEM),
            pl.BlockSpec(memory_space=pltpu.SMEM),
            pl.BlockSpec(memory_space=pltpu.SMEM),
            pl.BlockSpec(memory_space=pltpu.VMEM),
            pl.BlockSpec(memory_space=pltpu.VMEM),
            pl.BlockSpec(memory_space=pltpu.VMEM),
        ],
        out_specs=[
            pl.BlockSpec(memory_space=pltpu.VMEM),
            pl.BlockSpec(memory_space=pltpu.VMEM),
        ],
        scratch_shapes=[
            pltpu.VMEM((N_PAD, 2 * DIM), f32),
            pltpu.VMEM((CB, 2 * DIM), f32),
            pltpu.VMEM((CB, 2 * DIM), f32),
            pltpu.VMEM((CB, 2 * DIM), f32),
            pltpu.VMEM((N_BUF,), jnp.int32),
            pltpu.SMEM((N_BUF,), jnp.int32),
            pltpu.SMEM((N_BUF,), jnp.int32),
            pltpu.SMEM((E,), jnp.int32),
            pltpu.SMEM(((E + 2 + 127) // 128 * 128,), jnp.int32),
            pltpu.SMEM((E + CB,), jnp.int32),
            pltpu.SMEM((E + CB,), jnp.int32),
            pltpu.SMEM((E + CB,), jnp.int32),
            pltpu.SemaphoreType.DMA,
        ],
    )(p_arr, l_arr, r_arr, roots_c, leaf_emb, wt, bb)

    return rootout[0], hout


# speedup vs baseline: 1.0479x; 1.0479x over previous
import jax
import jax.numpy as jnp
from jax import lax
from jax.experimental import pallas as pl
from jax.experimental.pallas import tpu as pltpu

N_BUF = 19999
N_PAD = 20000
DIM = 128
CB = 256
G8 = 8


def _al8(i):
    return pl.multiple_of((i >> 3) << 3, 8)


def _tree_lstm_kernel(ps, ls, rs, roots, leaf, wt, bb,
                      hout, rootout,
                      hc, xl, xr, yb, zv, ll, lr, lvl, off, sp, sl, sr, sem):
    E = ps.shape[0]
    NRES = leaf.shape[0]
    f32 = jnp.float32
    sub_iota = lax.broadcasted_iota(jnp.int32, (G8, 1), 0)

    def pick_row(tile, idx):
        m = sub_iota == (idx & 7)
        return jnp.sum(jnp.where(m, tile, 0.0), axis=0, keepdims=True)

    zv[...] = jnp.zeros((N_BUF,), jnp.int32)
    cp_ll = pltpu.make_async_copy(zv, ll, sem)
    cp_lr = pltpu.make_async_copy(zv, lr, sem)
    off_n = (E + 2 + 127) // 128 * 128
    cp_off = pltpu.make_async_copy(zv.at[pl.ds(0, off_n)], off, sem)
    cp_ll.start()
    cp_lr.start()
    cp_off.start()

    hc[...] = jnp.zeros((N_PAD, 2 * DIM), f32)
    hc[0:NRES, 0:DIM] = leaf[...]

    cp_ll.wait()
    cp_lr.wait()
    cp_off.wait()

    def edge_level(e):
        le = ls[e]
        re_ = rs[e]
        pe = ps[e]
        lv = jnp.maximum(jnp.maximum(ll[le], ll[re_]) + 1,
                         jnp.maximum(ll[pe], lr[pe]))
        lvl[e] = lv
        lr[le] = jnp.maximum(lr[le], lv)
        lr[re_] = jnp.maximum(lr[re_], lv)
        ll[pe] = lv
        off[lv] = off[lv] + 1
        return lv

    UNR = 12

    def pass1(i, nlev):
        e = i * UNR
        m = nlev
        for u in range(UNR):
            m = jnp.maximum(m, edge_level(e + u))
        return m
    nlev = lax.fori_loop(0, E // UNR, pass1, jnp.int32(0))
    for e_tail in range((E // UNR) * UNR, E):
        nlev = jnp.maximum(nlev, edge_level(e_tail))

    def pass2(L, acc):
        cnt = off[L]
        off[L] = acc
        return acc + cnt
    lax.fori_loop(1, nlev + 1, pass2, jnp.int32(0))

    def edge_place(e):
        lv = lvl[e]
        pos = off[lv]
        off[lv] = pos + 1
        sp[pos] = ps[e]
        sl[pos] = ls[e]
        sr[pos] = rs[e]

    def pass3(i, c):
        e = i * UNR
        for u in range(UNR):
            edge_place(e + u)
        return c
    lax.fori_loop(0, E // UNR, pass3, 0)
    for e_tail in range((E // UNR) * UNR, E):
        edge_place(e_tail)

    def level_body(L, c):
        start = off[L - 1]
        end = off[L]
        nchunks = (end - start + CB - 1) // CB

        def chunk_body(ck, c2):
            base = start + ck * CB
            ngroups = (jnp.minimum(CB, end - base) + G8 - 1) // G8

            def gather_group(g, c3):
                gb = pl.multiple_of(g * G8, G8)
                rows_l = []
                rows_r = []
                for m in range(G8):
                    pos = base + gb + m
                    ok = pos < end
                    il = jnp.where(ok, sl[pos], 0)
                    ir = jnp.where(ok, sr[pos], 0)
                    rows_l.append(pick_row(hc[pl.ds(_al8(il), G8), :], il))
                    rows_r.append(pick_row(hc[pl.ds(_al8(ir), G8), :], ir))
                xl[pl.ds(gb, G8), :] = jnp.concatenate(rows_l, axis=0)
                xr[pl.ds(gb, G8), :] = jnp.concatenate(rows_r, axis=0)
                return c3
            lax.fori_loop(0, ngroups, gather_group, 0)

            xlv = xl[...]
            xrv = xr[...]
            comb = jnp.concatenate([xlv[:, 0:DIM], xrv[:, 0:DIM]], axis=1)
            out = jnp.dot(comb, wt[...], preferred_element_type=f32) + bb[...]
            i_g = jax.nn.sigmoid(out[:, 0:DIM])
            f_l = jax.nn.sigmoid(out[:, DIM:2 * DIM])
            f_r = jax.nn.sigmoid(out[:, 2 * DIM:3 * DIM])
            o_g = jax.nn.sigmoid(out[:, 3 * DIM:4 * DIM])
            g_g = jnp.tanh(out[:, 4 * DIM:5 * DIM])
            c_new = f_l * xlv[:, DIM:2 * DIM] + f_r * xrv[:, DIM:2 * DIM] + i_g * g_g
            h_new = o_g * jnp.tanh(c_new)
            yb[...] = jnp.concatenate([h_new, c_new], axis=1)

            def scatter_group(g, c3):
                gb = pl.multiple_of(g * G8, G8)
                yt = yb[pl.ds(gb, G8), :]
                for m in range(G8):
                    pos = base + gb + m
                    ok = pos < end
                    ip = jnp.where(ok, sp[pos], 0)
                    tb = _al8(ip)
                    tile = hc[pl.ds(tb, G8), :]
                    sel = jnp.logical_and(sub_iota == (ip & 7), ok)
                    row = jnp.broadcast_to(yt[m:m + 1, :], (G8, 2 * DIM))
                    hc[pl.ds(tb, G8), :] = jnp.where(sel, row, tile)
                return c3
            lax.fori_loop(0, ngroups, scatter_group, 0)
            return c2
        lax.fori_loop(0, nchunks, chunk_body, 0)
        return c
    lax.fori_loop(1, nlev + 1, level_body, 0)

    acc = jnp.zeros((1, DIM), f32)
    for k in range(4):
        rt = roots[k]
        acc = acc + pick_row(hc[pl.ds(_al8(rt), G8), 0:DIM], rt)
    rootout[:, 0:DIM] = jnp.broadcast_to(acc * 0.25, (G8, DIM))
    rootout[:, DIM:2 * DIM] = jnp.zeros((G8, DIM), f32)
    hout[:, 0:DIM] = hc[0:NRES, 0:DIM]
    hout[:, DIM:2 * DIM] = jnp.zeros((NRES, DIM), f32)


def kernel(leaf_emb, forest_edges, n_res, forest_roots, W_up, b_up, W_down, b_down):
    f32 = jnp.float32
    NRES, dim = leaf_emb.shape
    E = forest_edges.shape[0]

    max_node = jnp.maximum(jnp.max(forest_edges[:, 0]), n_res - 1)
    p_arr = forest_edges[:, 0]
    l_arr = jnp.minimum(forest_edges[:, 1], max_node)
    r_arr = jnp.minimum(forest_edges[:, 2], max_node)
    roots_c = jnp.minimum(forest_roots, max_node).astype(jnp.int32)

    wt = jnp.transpose(W_up)
    bb = b_up.reshape(1, 5 * dim)

    hout, rootout = pl.pallas_call(
        _tree_lstm_kernel,
        out_shape=[
            jax.ShapeDtypeStruct((NRES, 2 * dim), f32),
            jax.ShapeDtypeStruct((G8, 2 * dim), f32),
        ],
        in_specs=[
            pl.BlockSpec(memory_space=pltpu.SMEM),
            pl.BlockSpec(memory_space=pltpu.SMEM),
            pl.BlockSpec(memory_space=pltpu.SMEM),
            pl.BlockSpec(memory_space=pltpu.SMEM),
            pl.BlockSpec(memory_space=pltpu.VMEM),
            pl.BlockSpec(memory_space=pltpu.VMEM),
            pl.BlockSpec(memory_space=pltpu.VMEM),
        ],
        out_specs=[
            pl.BlockSpec(memory_space=pltpu.VMEM),
            pl.BlockSpec(memory_space=pltpu.VMEM),
        ],
        scratch_shapes=[
            pltpu.VMEM((N_PAD, 2 * DIM), f32),
            pltpu.VMEM((CB, 2 * DIM), f32),
            pltpu.VMEM((CB, 2 * DIM), f32),
            pltpu.VMEM((CB, 2 * DIM), f32),
            pltpu.VMEM((N_BUF,), jnp.int32),
            pltpu.SMEM((N_BUF,), jnp.int32),
            pltpu.SMEM((N_BUF,), jnp.int32),
            pltpu.SMEM((E,), jnp.int32),
            pltpu.SMEM(((E + 2 + 127) // 128 * 128,), jnp.int32),
            pltpu.SMEM((E + CB,), jnp.int32),
            pltpu.SMEM((E + CB,), jnp.int32),
            pltpu.SMEM((E + CB,), jnp.int32),
            pltpu.SemaphoreType.DMA,
        ],
    )(p_arr, l_arr, r_arr, roots_c, leaf_emb, wt, bb)

    return rootout[0], hout


# speedup vs baseline: 1.0605x; 1.0120x over previous
import jax
import jax.numpy as jnp
from jax import lax
from jax.experimental import pallas as pl
from jax.experimental.pallas import tpu as pltpu

N_BUF = 19999
N_PAD = 20000
DIM = 128
CB = 256
G8 = 8


def _al8(i):
    return pl.multiple_of((i >> 3) << 3, 8)


def _tree_lstm_kernel(ps, ls, rs, roots, leaf, wt, bb,
                      hout, rootout,
                      hc, xl, xr, yb, zv, ll, lr, lvl, off, sp, sl, sr, sem):
    E = ps.shape[0]
    NRES = leaf.shape[0]
    f32 = jnp.float32
    sub_iota = lax.broadcasted_iota(jnp.int32, (G8, 1), 0)

    def pick_row(tile, idx):
        m = sub_iota == (idx & 7)
        return jnp.sum(jnp.where(m, tile, 0.0), axis=0, keepdims=True)

    zv[...] = jnp.zeros((N_BUF,), jnp.int32)
    cp_ll = pltpu.make_async_copy(zv, ll, sem)
    cp_lr = pltpu.make_async_copy(zv, lr, sem)
    off_n = (E + 2 + 127) // 128 * 128
    cp_off = pltpu.make_async_copy(zv.at[pl.ds(0, off_n)], off, sem)
    cp_ll.start()
    cp_lr.start()
    cp_off.start()

    hc[...] = jnp.zeros((N_PAD, 2 * DIM), f32)
    hc[0:NRES, 0:DIM] = leaf[...]

    cp_ll.wait()
    cp_lr.wait()
    cp_off.wait()

    def edge_level(e):
        le = ls[e]
        re_ = rs[e]
        pe = ps[e]
        lv = jnp.maximum(jnp.maximum(ll[le], ll[re_]) + 1,
                         jnp.maximum(ll[pe], lr[pe]))
        lvl[e] = lv
        lr[le] = jnp.maximum(lr[le], lv)
        lr[re_] = jnp.maximum(lr[re_], lv)
        ll[pe] = lv
        off[lv] = off[lv] + 1
        return lv

    UNR = 24

    def pass1(i, nlev):
        e = i * UNR
        m = nlev
        for u in range(UNR):
            m = jnp.maximum(m, edge_level(e + u))
        return m
    nlev = lax.fori_loop(0, E // UNR, pass1, jnp.int32(0))
    for e_tail in range((E // UNR) * UNR, E):
        nlev = jnp.maximum(nlev, edge_level(e_tail))

    def pass2(L, acc):
        cnt = off[L]
        off[L] = acc
        return acc + cnt
    lax.fori_loop(1, nlev + 1, pass2, jnp.int32(0))

    def edge_place(e):
        lv = lvl[e]
        pos = off[lv]
        off[lv] = pos + 1
        sp[pos] = ps[e]
        sl[pos] = ls[e]
        sr[pos] = rs[e]

    def pass3(i, c):
        e = i * UNR
        for u in range(UNR):
            edge_place(e + u)
        return c
    lax.fori_loop(0, E // UNR, pass3, 0)
    for e_tail in range((E // UNR) * UNR, E):
        edge_place(e_tail)

    def level_body(L, c):
        start = off[L - 1]
        end = off[L]
        nchunks = (end - start + CB - 1) // CB

        def chunk_body(ck, c2):
            base = start + ck * CB
            ngroups = (jnp.minimum(CB, end - base) + G8 - 1) // G8

            def gather_group(g, c3):
                gb = pl.multiple_of(g * G8, G8)
                rows_l = []
                rows_r = []
                for m in range(G8):
                    pos = base + gb + m
                    ok = pos < end
                    il = jnp.where(ok, sl[pos], 0)
                    ir = jnp.where(ok, sr[pos], 0)
                    rows_l.append(pick_row(hc[pl.ds(_al8(il), G8), :], il))
                    rows_r.append(pick_row(hc[pl.ds(_al8(ir), G8), :], ir))
                xl[pl.ds(gb, G8), :] = jnp.concatenate(rows_l, axis=0)
                xr[pl.ds(gb, G8), :] = jnp.concatenate(rows_r, axis=0)
                return c3
            lax.fori_loop(0, ngroups, gather_group, 0)

            xlv = xl[...]
            xrv = xr[...]
            comb = jnp.concatenate([xlv[:, 0:DIM], xrv[:, 0:DIM]], axis=1)
            out = jnp.dot(comb, wt[...], preferred_element_type=f32) + bb[...]
            i_g = jax.nn.sigmoid(out[:, 0:DIM])
            f_l = jax.nn.sigmoid(out[:, DIM:2 * DIM])
            f_r = jax.nn.sigmoid(out[:, 2 * DIM:3 * DIM])
            o_g = jax.nn.sigmoid(out[:, 3 * DIM:4 * DIM])
            g_g = jnp.tanh(out[:, 4 * DIM:5 * DIM])
            c_new = f_l * xlv[:, DIM:2 * DIM] + f_r * xrv[:, DIM:2 * DIM] + i_g * g_g
            h_new = o_g * jnp.tanh(c_new)
            yb[...] = jnp.concatenate([h_new, c_new], axis=1)

            def scatter_group(g, c3):
                gb = pl.multiple_of(g * G8, G8)
                yt = yb[pl.ds(gb, G8), :]
                for m in range(G8):
                    pos = base + gb + m
                    ok = pos < end
                    ip = jnp.where(ok, sp[pos], 0)
                    tb = _al8(ip)
                    tile = hc[pl.ds(tb, G8), :]
                    sel = jnp.logical_and(sub_iota == (ip & 7), ok)
                    row = jnp.broadcast_to(yt[m:m + 1, :], (G8, 2 * DIM))
                    hc[pl.ds(tb, G8), :] = jnp.where(sel, row, tile)
                return c3
            lax.fori_loop(0, ngroups, scatter_group, 0)
            return c2
        lax.fori_loop(0, nchunks, chunk_body, 0)
        return c
    lax.fori_loop(1, nlev + 1, level_body, 0)

    acc = jnp.zeros((1, DIM), f32)
    for k in range(4):
        rt = roots[k]
        acc = acc + pick_row(hc[pl.ds(_al8(rt), G8), 0:DIM], rt)
    rootout[:, 0:DIM] = jnp.broadcast_to(acc * 0.25, (G8, DIM))
    rootout[:, DIM:2 * DIM] = jnp.zeros((G8, DIM), f32)
    hout[:, 0:DIM] = hc[0:NRES, 0:DIM]
    hout[:, DIM:2 * DIM] = jnp.zeros((NRES, DIM), f32)


def kernel(leaf_emb, forest_edges, n_res, forest_roots, W_up, b_up, W_down, b_down):
    f32 = jnp.float32
    NRES, dim = leaf_emb.shape
    E = forest_edges.shape[0]

    max_node = jnp.maximum(jnp.max(forest_edges[:, 0]), n_res - 1)
    p_arr = forest_edges[:, 0]
    l_arr = jnp.minimum(forest_edges[:, 1], max_node)
    r_arr = jnp.minimum(forest_edges[:, 2], max_node)
    roots_c = jnp.minimum(forest_roots, max_node).astype(jnp.int32)

    wt = jnp.transpose(W_up)
    bb = b_up.reshape(1, 5 * dim)

    hout, rootout = pl.pallas_call(
        _tree_lstm_kernel,
        out_shape=[
            jax.ShapeDtypeStruct((NRES, 2 * dim), f32),
            jax.ShapeDtypeStruct((G8, 2 * dim), f32),
        ],
        in_specs=[
            pl.BlockSpec(memory_space=pltpu.SMEM),
            pl.BlockSpec(memory_space=pltpu.SMEM),
            pl.BlockSpec(memory_space=pltpu.SMEM),
            pl.BlockSpec(memory_space=pltpu.SMEM),
            pl.BlockSpec(memory_space=pltpu.VMEM),
            pl.BlockSpec(memory_space=pltpu.VMEM),
            pl.BlockSpec(memory_space=pltpu.VMEM),
        ],
        out_specs=[
            pl.BlockSpec(memory_space=pltpu.VMEM),
            pl.BlockSpec(memory_space=pltpu.VMEM),
        ],
        scratch_shapes=[
            pltpu.VMEM((N_PAD, 2 * DIM), f32),
            pltpu.VMEM((CB, 2 * DIM), f32),
            pltpu.VMEM((CB, 2 * DIM), f32),
            pltpu.VMEM((CB, 2 * DIM), f32),
            pltpu.VMEM((N_BUF,), jnp.int32),
            pltpu.SMEM((N_BUF,), jnp.int32),
            pltpu.SMEM((N_BUF,), jnp.int32),
            pltpu.SMEM((E,), jnp.int32),
            pltpu.SMEM(((E + 2 + 127) // 128 * 128,), jnp.int32),
            pltpu.SMEM((E + CB,), jnp.int32),
            pltpu.SMEM((E + CB,), jnp.int32),
            pltpu.SMEM((E + CB,), jnp.int32),
            pltpu.SemaphoreType.DMA,
        ],
    )(p_arr, l_arr, r_arr, roots_c, leaf_emb, wt, bb)

    return rootout[0], hout
